# baseline (speedup 1.0000x reference)
import functools

import jax
import jax.numpy as jnp
from jax import lax
from jax.experimental import pallas as pl
from jax.experimental.pallas import tpu as pltpu
from jax.experimental.pallas import tpu_sc as plsc

B = 8
HW = 1024
C = 512
N = 512
K = 48
KTOP = 24
ROWS = B * N
NC, NS = 2, 16
NW = NC * NS
RPW = ROWS // NW

_F32 = jnp.float32
_I32 = jnp.int32
_HIGHEST = lax.Precision.HIGHEST


def _gram_body(f_ref, s_ref):
    x = f_ref[...]
    ss = jnp.sum(x * x, axis=1, keepdims=True)
    nrm = jnp.maximum(jnp.sqrt(ss), 1e-12)
    xn = x / nrm
    s_ref[...] = lax.dot_general(
        xn, xn, (((1,), (1,)), ((), ())),
        preferred_element_type=_F32)


def _gram(feat_r):
    return pl.pallas_call(
        _gram_body,
        grid=(B,),
        in_specs=[pl.BlockSpec((None, HW, C), lambda i: (i, 0, 0))],
        out_specs=pl.BlockSpec((None, HW, HW), lambda i: (i, 0, 0)),
        out_shape=jax.ShapeDtypeStruct((B, HW, HW), _F32),
    )(feat_r)


_SUBCH = 32


def _simgather_body(s_ref, pid_ref, lidT_ref, simT_ref,
                    lT, pref, idxb, rows_v, sbufT, sem0, sem1):
    wid = lax.axis_index("s") * NC + lax.axis_index("c")
    b = wid // 4
    n0 = (wid % 4) * 128
    row0 = wid * RPW
    pltpu.sync_copy(lidT_ref.at[:, pl.ds(n0, 128)], lT)
    pltpu.sync_copy(pid_ref.at[pl.ds(n0, 128)], pref)
    iota16 = lax.iota(_I32, 16)
    sems = [sem0, sem1]
    nburst = RPW // _SUBCH

    def build(sc):
        for h in range(2):
            idxb[sc % 2, pl.ds(16 * h, 16)] = \
                pref[pl.ds(sc * _SUBCH + 16 * h, 16)] + b * HW

    def start(sc):
        return pltpu.async_copy(s_ref.at[idxb.at[sc % 2]],
                                rows_v.at[sc % 2], sems[sc % 2])

    build(0)
    descs = {0: start(0)}
    for sc in range(nburst):
        if sc + 1 < nburst:
            build(sc + 1)
            descs[sc + 1] = start(sc + 1)
        descs[sc].wait()
        for g in range(2):
            maj = iota16 + g * 16
            col = sc * _SUBCH + g * 16
            for j in range(K):
                lvec = lT[j, pl.ds(col, 16)]
                sbufT[j, pl.ds(col, 16)] = plsc.load_gather(
                    rows_v.at[sc % 2], [maj, lvec])
    pltpu.sync_copy(sbufT, simT_ref.at[:, pl.ds(row0, RPW)])


def _simgather(s_rows, patch_id, local_id_T):
    mesh = plsc.VectorSubcoreMesh(
        core_axis_name="c", subcore_axis_name="s",
        num_cores=NC, num_subcores=NS)
    fn = pl.kernel(
        _simgather_body,
        out_type=jax.ShapeDtypeStruct((K, ROWS), _F32),
        mesh=mesh,
        compiler_params=pltpu.CompilerParams(needs_layout_passes=False),
        scratch_types=[
            pltpu.VMEM((K, 128), _I32),
            pltpu.VMEM((128,), _I32),
            pltpu.VMEM((2, _SUBCH), _I32),
            pltpu.VMEM((2, _SUBCH, HW), _F32),
            pltpu.VMEM((K, 128), _F32),
            pltpu.SemaphoreType.DMA,
            pltpu.SemaphoreType.DMA,
        ],
    )
    return fn(s_rows, patch_id, local_id_T)


def _topk_body(sim_ref, locT_ref, pid_ref, top_ref, posT_ref):
    s = sim_ref[...]
    locT = locT_ref[...]
    si = lax.broadcasted_iota(_I32, (K, N), 0)
    rank = jnp.zeros((K, N), _I32)
    for j in range(K):
        sj = s[j:j + 1, :]
        beats = (sj > s) | ((sj == s) & (si > j))
        rank = rank + beats.astype(_I32)

    rows_top, rows_sel = [], []
    for r in range(KTOP):
        m = rank == r
        rows_top.append(jnp.sum(jnp.where(m, si, 0), axis=0, keepdims=True))
        rows_sel.append(jnp.sum(jnp.where(m, locT, 0), axis=0, keepdims=True))
    top_ref[...] = jnp.transpose(jnp.concatenate(rows_top, axis=0))
    posT_ref[...] = jnp.concatenate([pid_ref[...]] + rows_sel, axis=0)


def _topk(sim_T, local_id_T, patch_id):
    return pl.pallas_call(
        _topk_body,
        grid=(B,),
        in_specs=[
            pl.BlockSpec((K, N), lambda i: (0, i)),
            pl.BlockSpec((K, N), lambda i: (0, 0)),
            pl.BlockSpec((1, N), lambda i: (0, 0)),
        ],
        out_specs=[
            pl.BlockSpec((N, KTOP), lambda i: (i, 0)),
            pl.BlockSpec((KTOP + 1, N), lambda i: (0, i)),
        ],
        out_shape=[
            jax.ShapeDtypeStruct((ROWS, KTOP), _I32),
            jax.ShapeDtypeStruct((KTOP + 1, ROWS), _I32),
        ],
    )(sim_T, local_id_T, patch_id)


_HALF = 64


def _scatter_body(posT_ref, z_ref, wt_ref, pbuf, chunk, sem):
    wid = lax.axis_index("s") * NC + lax.axis_index("c")
    row0 = wid * RPW
    pltpu.sync_copy(posT_ref.at[:, pl.ds(row0, RPW)], pbuf)
    iota16 = lax.iota(_I32, 16)
    ones = jnp.ones((16,), _F32)
    for half in range(RPW // _HALF):
        pltpu.sync_copy(z_ref, chunk)
        for g in range(_HALF // 16):
            rvec = g * 16 + iota16
            for j in range(KTOP + 1):
                pv = pbuf[j, pl.ds(half * _HALF + g * 16, 16)]
                plsc.addupdate_scatter(chunk, [rvec, pv], ones)
        pltpu.sync_copy(chunk, wt_ref.at[pl.ds(row0 + half * _HALF, _HALF)])


def _scatter(pos_T, zeros_hbm):
    mesh = plsc.VectorSubcoreMesh(
        core_axis_name="c", subcore_axis_name="s",
        num_cores=NC, num_subcores=NS)
    fn = pl.kernel(
        _scatter_body,
        out_type=jax.ShapeDtypeStruct((ROWS, HW), _F32),
        mesh=mesh,
        compiler_params=pltpu.CompilerParams(needs_layout_passes=False),
        scratch_types=[
            pltpu.VMEM((KTOP + 1, RPW), _I32),
            pltpu.VMEM((_HALF, HW), _F32),
            pltpu.SemaphoreType.DMA,
        ],
    )
    return fn(pos_T, zeros_hbm)


def _mlp_body(wt_ref, f_ref, w1_ref, b1_ref, g1_ref, be1_ref, w2_ref,
              b2_ref, o_ref):
    xs = lax.dot_general(
        wt_ref[...], f_ref[...], (((1,), (0,)), ((), ())),
        preferred_element_type=_F32) * (1.0 / 25.0)
    h1 = lax.dot_general(
        xs, w1_ref[...], (((1,), (0,)), ((), ())),
        preferred_element_type=_F32) + b1_ref[...]
    h1 = (h1 / jnp.sqrt(jnp.float32(1.0 + 1e-5))) * g1_ref[...] + be1_ref[...]
    h1 = jnp.maximum(h1, 0.0)
    o_ref[...] = lax.dot_general(
        h1, w2_ref[...], (((1,), (0,)), ((), ())),
        preferred_element_type=_F32) + b2_ref[...]


def _mlp(wt, feat_r, W1, b1, gamma1, beta1, W2, b2):
    return pl.pallas_call(
        _mlp_body,
        grid=(B,),
        in_specs=[
            pl.BlockSpec((None, N, HW), lambda i: (i, 0, 0)),
            pl.BlockSpec((None, HW, C), lambda i: (i, 0, 0)),
            pl.BlockSpec((C, 1024), lambda i: (0, 0)),
            pl.BlockSpec((1, 1024), lambda i: (0, 0)),
            pl.BlockSpec((1, 1024), lambda i: (0, 0)),
            pl.BlockSpec((1, 1024), lambda i: (0, 0)),
            pl.BlockSpec((1024, 256), lambda i: (0, 0)),
            pl.BlockSpec((1, 256), lambda i: (0, 0)),
        ],
        out_specs=pl.BlockSpec((None, N, 256), lambda i: (i, 0, 0)),
        out_shape=jax.ShapeDtypeStruct((B, N, 256), _F32),
    )(wt, feat_r, W1, b1, gamma1, beta1, W2, b2)


def kernel(patch_size, feats, num_patches, patch_ids, patch_local_ids,
           W1, b1, gamma1, beta1, W2, b2):
    feat = feats[0]
    feat_r = jnp.transpose(feat, (0, 2, 3, 1)).reshape(B, HW, C)
    patch_id = patch_ids[0][:, 0] if patch_ids.ndim == 3 else patch_ids[0]
    local_id = patch_local_ids[0]

    S = _gram(feat_r)
    local_T = jnp.transpose(local_id.astype(_I32))
    sim_T = _simgather(S.reshape(B * HW, HW), patch_id.astype(_I32), local_T)
    top_idx, pos_T = _topk(sim_T, local_T,
                           patch_id.astype(_I32).reshape(1, N))
    zeros_hbm = jnp.zeros((_HALF, HW), _F32)
    wt = _scatter(pos_T, zeros_hbm)
    out = _mlp(wt.reshape(B, N, HW), feat_r, W1,
               b1.reshape(1, 1024), gamma1.reshape(1, 1024),
               beta1.reshape(1, 1024), W2, b2.reshape(1, 256))

    return (out.reshape(ROWS, 256), patch_id, local_id,
            top_idx.reshape(ROWS, KTOP, 1))


# speedup vs baseline: 1.0104x; 1.0104x over previous
import functools

import jax
import jax.numpy as jnp
from jax import lax
from jax.experimental import pallas as pl
from jax.experimental.pallas import tpu as pltpu
from jax.experimental.pallas import tpu_sc as plsc

B = 8
HW = 1024
C = 512
N = 512
K = 48
KTOP = 24
ROWS = B * N
NC, NS = 2, 16
NW = NC * NS
RPW = ROWS // NW

_F32 = jnp.float32
_I32 = jnp.int32
_HIGHEST = lax.Precision.HIGHEST


def _gram_body(f_ref, s_ref):
    x = f_ref[...]
    ss = jnp.sum(x * x, axis=1, keepdims=True)
    nrm = jnp.maximum(jnp.sqrt(ss), 1e-12)
    xn = x / nrm
    s_ref[...] = lax.dot_general(
        xn, xn, (((1,), (1,)), ((), ())),
        preferred_element_type=_F32)


def _gram(feat_r):
    return pl.pallas_call(
        _gram_body,
        grid=(B,),
        in_specs=[pl.BlockSpec((None, HW, C), lambda i: (i, 0, 0))],
        out_specs=pl.BlockSpec((None, HW, HW), lambda i: (i, 0, 0)),
        out_shape=jax.ShapeDtypeStruct((B, HW, HW), _F32),
    )(feat_r)


_SUBCH = 32
DEPTH = 3


def _simgather_body(s_ref, pid_ref, lidT_ref, simT_ref,
                    lT, pref, idxb, rows_v, sbufT, sem0, sem1, sem2):
    wid = lax.axis_index("s") * NC + lax.axis_index("c")
    b = wid // 4
    n0 = (wid % 4) * 128
    row0 = wid * RPW
    pltpu.sync_copy(lidT_ref.at[:, pl.ds(n0, 128)], lT)
    pltpu.sync_copy(pid_ref.at[pl.ds(n0, 128)], pref)
    iota16 = lax.iota(_I32, 16)
    sems = [sem0, sem1, sem2]
    nburst = RPW // _SUBCH

    def build(sc):
        for h in range(2):
            idxb[sc % DEPTH, pl.ds(16 * h, 16)] = \
                pref[pl.ds(sc * _SUBCH + 16 * h, 16)] + b * HW

    def start(sc):
        return pltpu.async_copy(s_ref.at[idxb.at[sc % DEPTH]],
                                rows_v.at[sc % DEPTH], sems[sc % DEPTH])

    for p0 in range(DEPTH - 1):
        build(p0)
    descs = {p0: start(p0) for p0 in range(DEPTH - 1)}
    for sc in range(nburst):
        if sc + DEPTH - 1 < nburst:
            build(sc + DEPTH - 1)
            descs[sc + DEPTH - 1] = start(sc + DEPTH - 1)
        descs[sc].wait()
        for g in range(2):
            maj = iota16 + g * 16
            col = sc * _SUBCH + g * 16
            for j in range(K):
                lvec = lT[j, pl.ds(col, 16)]
                sbufT[j, pl.ds(col, 16)] = plsc.load_gather(
                    rows_v.at[sc % DEPTH], [maj, lvec])
    pltpu.sync_copy(sbufT, simT_ref.at[:, pl.ds(row0, RPW)])


def _simgather(s_rows, patch_id, local_id_T):
    mesh = plsc.VectorSubcoreMesh(
        core_axis_name="c", subcore_axis_name="s",
        num_cores=NC, num_subcores=NS)
    fn = pl.kernel(
        _simgather_body,
        out_type=jax.ShapeDtypeStruct((K, ROWS), _F32),
        mesh=mesh,
        compiler_params=pltpu.CompilerParams(needs_layout_passes=False),
        scratch_types=[
            pltpu.VMEM((K, 128), _I32),
            pltpu.VMEM((128,), _I32),
            pltpu.VMEM((DEPTH, _SUBCH), _I32),
            pltpu.VMEM((DEPTH, _SUBCH, HW), _F32),
            pltpu.VMEM((K, 128), _F32),
            pltpu.SemaphoreType.DMA,
            pltpu.SemaphoreType.DMA,
            pltpu.SemaphoreType.DMA,
        ],
    )
    return fn(s_rows, patch_id, local_id_T)


def _topk_body(sim_ref, locT_ref, pid_ref, top_ref, posT_ref):
    s = sim_ref[...]
    locT = locT_ref[...]
    si = lax.broadcasted_iota(_I32, (K, N), 0)
    rank = jnp.zeros((K, N), _I32)
    for j in range(K):
        sj = s[j:j + 1, :]
        beats = (sj > s) | ((sj == s) & (si > j))
        rank = rank + beats.astype(_I32)

    rows_top, rows_sel = [], []
    for r in range(KTOP):
        m = rank == r
        rows_top.append(jnp.sum(jnp.where(m, si, 0), axis=0, keepdims=True))
        rows_sel.append(jnp.sum(jnp.where(m, locT, 0), axis=0, keepdims=True))
    top_ref[...] = jnp.transpose(jnp.concatenate(rows_top, axis=0))
    posT_ref[...] = jnp.concatenate([pid_ref[...]] + rows_sel, axis=0)


def _topk(sim_T, local_id_T, patch_id):
    return pl.pallas_call(
        _topk_body,
        grid=(B,),
        in_specs=[
            pl.BlockSpec((K, N), lambda i: (0, i)),
            pl.BlockSpec((K, N), lambda i: (0, 0)),
            pl.BlockSpec((1, N), lambda i: (0, 0)),
        ],
        out_specs=[
            pl.BlockSpec((N, KTOP), lambda i: (i, 0)),
            pl.BlockSpec((KTOP + 1, N), lambda i: (0, i)),
        ],
        out_shape=[
            jax.ShapeDtypeStruct((ROWS, KTOP), _I32),
            jax.ShapeDtypeStruct((KTOP + 1, ROWS), _I32),
        ],
    )(sim_T, local_id_T, patch_id)


_HALF = 64


def _scatter_body(posT_ref, z_ref, wt_ref, pbuf, chunk, sem):
    wid = lax.axis_index("s") * NC + lax.axis_index("c")
    row0 = wid * RPW
    pltpu.sync_copy(posT_ref.at[:, pl.ds(row0, RPW)], pbuf)
    iota16 = lax.iota(_I32, 16)
    ones = jnp.ones((16,), _F32)
    for half in range(RPW // _HALF):
        pltpu.sync_copy(z_ref, chunk)
        for g in range(_HALF // 16):
            rvec = g * 16 + iota16
            for j in range(KTOP + 1):
                pv = pbuf[j, pl.ds(half * _HALF + g * 16, 16)]
                plsc.addupdate_scatter(chunk, [rvec, pv], ones)
        pltpu.sync_copy(chunk, wt_ref.at[pl.ds(row0 + half * _HALF, _HALF)])


def _scatter(pos_T, zeros_hbm):
    mesh = plsc.VectorSubcoreMesh(
        core_axis_name="c", subcore_axis_name="s",
        num_cores=NC, num_subcores=NS)
    fn = pl.kernel(
        _scatter_body,
        out_type=jax.ShapeDtypeStruct((ROWS, HW), _F32),
        mesh=mesh,
        compiler_params=pltpu.CompilerParams(needs_layout_passes=False),
        scratch_types=[
            pltpu.VMEM((KTOP + 1, RPW), _I32),
            pltpu.VMEM((_HALF, HW), _F32),
            pltpu.SemaphoreType.DMA,
        ],
    )
    return fn(pos_T, zeros_hbm)


def _mlp_body(wt_ref, f_ref, w1_ref, b1_ref, g1_ref, be1_ref, w2_ref,
              b2_ref, o_ref):
    xs = lax.dot_general(
        wt_ref[...], f_ref[...], (((1,), (0,)), ((), ())),
        preferred_element_type=_F32) * (1.0 / 25.0)
    h1 = lax.dot_general(
        xs, w1_ref[...], (((1,), (0,)), ((), ())),
        preferred_element_type=_F32) + b1_ref[...]
    h1 = (h1 / jnp.sqrt(jnp.float32(1.0 + 1e-5))) * g1_ref[...] + be1_ref[...]
    h1 = jnp.maximum(h1, 0.0)
    o_ref[...] = lax.dot_general(
        h1, w2_ref[...], (((1,), (0,)), ((), ())),
        preferred_element_type=_F32) + b2_ref[...]


def _mlp(wt, feat_r, W1, b1, gamma1, beta1, W2, b2):
    return pl.pallas_call(
        _mlp_body,
        grid=(B,),
        in_specs=[
            pl.BlockSpec((None, N, HW), lambda i: (i, 0, 0)),
            pl.BlockSpec((None, HW, C), lambda i: (i, 0, 0)),
            pl.BlockSpec((C, 1024), lambda i: (0, 0)),
            pl.BlockSpec((1, 1024), lambda i: (0, 0)),
            pl.BlockSpec((1, 1024), lambda i: (0, 0)),
            pl.BlockSpec((1, 1024), lambda i: (0, 0)),
            pl.BlockSpec((1024, 256), lambda i: (0, 0)),
            pl.BlockSpec((1, 256), lambda i: (0, 0)),
        ],
        out_specs=pl.BlockSpec((None, N, 256), lambda i: (i, 0, 0)),
        out_shape=jax.ShapeDtypeStruct((B, N, 256), _F32),
    )(wt, feat_r, W1, b1, gamma1, beta1, W2, b2)


def kernel(patch_size, feats, num_patches, patch_ids, patch_local_ids,
           W1, b1, gamma1, beta1, W2, b2):
    feat = feats[0]
    feat_r = jnp.transpose(feat, (0, 2, 3, 1)).reshape(B, HW, C)
    patch_id = patch_ids[0][:, 0] if patch_ids.ndim == 3 else patch_ids[0]
    local_id = patch_local_ids[0]

    S = _gram(feat_r)
    local_T = jnp.transpose(local_id.astype(_I32))
    sim_T = _simgather(S.reshape(B * HW, HW), patch_id.astype(_I32), local_T)
    top_idx, pos_T = _topk(sim_T, local_T,
                           patch_id.astype(_I32).reshape(1, N))
    zeros_hbm = jnp.zeros((_HALF, HW), _F32)
    wt = _scatter(pos_T, zeros_hbm)
    out = _mlp(wt.reshape(B, N, HW), feat_r, W1,
               b1.reshape(1, 1024), gamma1.reshape(1, 1024),
               beta1.reshape(1, 1024), W2, b2.reshape(1, 256))

    return (out.reshape(ROWS, 256), patch_id, local_id,
            top_idx.reshape(ROWS, KTOP, 1))
